# baseline (speedup 1.0000x reference)
import functools

import jax
import jax.numpy as jnp
from jax import lax
from jax.experimental import pallas as pl
from jax.experimental.pallas import tpu as pltpu
from jax.experimental.pallas import tpu_sc as plsc

B = 4096
F = 26
C = 1000
OUT_W = F * C
NC, NS, L = 2, 16, 16
NW = NC * NS
ROWS = B // NW
ZCHUNK = 5


def _sc_one_hot_body(x_hbm, out_hbm, x_v, buf0, buf1, sem0, sem1):
    wid = lax.axis_index("s") * NC + lax.axis_index("c")
    base = wid * ROWS

    pltpu.sync_copy(x_hbm.at[pl.ds(base * F, ROWS * F)], x_v)

    lanes = lax.iota(jnp.int32, L)
    ones = jnp.full((L,), 1.0, jnp.float32)
    zeros = jnp.zeros((L,), jnp.float32)

    f0 = lanes
    f1 = jnp.minimum(lanes + 16, F - 1)
    m1 = lanes < (F - 16)

    def row_targets(r):
        rbase = jnp.full((L,), r * F, jnp.int32)
        xv0 = plsc.load_gather(x_v, [rbase + f0])
        xv1 = plsc.load_gather(x_v, [rbase + f1])
        return f0 * C + xv0, f1 * C + xv1

    def paint(buf, r, val):
        i0, i1 = row_targets(r)
        plsc.store_scatter(buf, [i0], val)
        plsc.store_scatter(buf, [i1], val, mask=m1)

    def zbody(k, _):
        o = k * (ZCHUNK * L)
        for j in range(ZCHUNK):
            buf0[pl.ds(o + j * L, L)] = zeros
            buf1[pl.ds(o + j * L, L)] = zeros
        return 0
    lax.fori_loop(0, OUT_W // (ZCHUNK * L), zbody, 0)

    bufs = (buf0, buf1)
    sems = (sem0, sem1)

    for b in range(2):
        paint(bufs[b], b, ones)
        pltpu.async_copy(bufs[b], out_hbm.at[base + b], sems[b])

    def body(j, _):
        for b in range(2):
            r = 2 * j + b
            pltpu.make_async_copy(bufs[b], out_hbm.at[base + r - 2],
                                  sems[b]).wait()
            paint(bufs[b], r - 2, zeros)
            paint(bufs[b], r, ones)
            pltpu.async_copy(bufs[b], out_hbm.at[base + r], sems[b])
        return 0
    lax.fori_loop(1, ROWS // 2, body, 0)

    for b in range(2):
        pltpu.make_async_copy(bufs[b], out_hbm.at[base + ROWS - 2 + b],
                              sems[b]).wait()


_sc_one_hot = functools.partial(
    pl.kernel,
    out_type=jax.ShapeDtypeStruct((B, OUT_W), jnp.float32),
    mesh=plsc.VectorSubcoreMesh(core_axis_name="c", subcore_axis_name="s"),
    compiler_params=pltpu.CompilerParams(needs_layout_passes=False),
    scratch_types=[
        pltpu.VMEM((ROWS * F,), jnp.int32),
        pltpu.VMEM((OUT_W,), jnp.float32),
        pltpu.VMEM((OUT_W,), jnp.float32),
        pltpu.SemaphoreType.DMA,
        pltpu.SemaphoreType.DMA,
    ],
)(_sc_one_hot_body)


@jax.jit
def kernel(x):
    return _sc_one_hot(x.astype(jnp.int32).reshape(B * F))
